# Initial kernel scaffold; baseline (speedup 1.0000x reference)
import jax
import jax.numpy as jnp
from jax.experimental import pallas as pl


def kernel(input_ids, state_table, species_table):
    raise NotImplementedError("write your pallas kernel here")


# speedup vs baseline: 4.2993x; 4.2993x over previous
import jax
import jax.numpy as jnp
from jax.experimental import pallas as pl


def _tc_body(ids_ref, state_ref, species_ref, out_ref):
    ids = ids_ref[...]
    st = state_ref[...]
    sp = species_ref[...]
    idsx = ids[:, :, None]
    r0 = st[0][None, None, :]
    r1 = st[1][None, None, :]
    r2 = st[2][None, None, :]
    state_emb = jnp.where(idsx == 0, r0, jnp.where(idsx == 1, r1, r2))
    out_ref[...] = state_emb + sp[:, None, :]


def kernel(input_ids, state_table, species_table):
    B, S, T = input_ids.shape
    H = state_table.shape[1]
    ids = input_ids.reshape(B * S, T).astype(jnp.int32)
    Sb = 32
    grid = (B * S) // Sb
    s_blocks = S // Sb
    out = pl.pallas_call(
        _tc_body,
        grid=(grid,),
        in_specs=[
            pl.BlockSpec((Sb, T), lambda i: (i, 0)),
            pl.BlockSpec((3, H), lambda i: (0, 0)),
            pl.BlockSpec((Sb, H), lambda i: (i % s_blocks, 0)),
        ],
        out_specs=pl.BlockSpec((Sb, T, H), lambda i: (i, 0, 0)),
        out_shape=jax.ShapeDtypeStruct((B * S, T, H), jnp.float32),
    )(ids, state_table, species_table)
    return out.reshape(B, S, T, H)
